# baseline (speedup 1.0000x reference)
import functools

import jax
import jax.numpy as jnp
from jax import lax
from jax.experimental import pallas as pl
from jax.experimental.pallas import tpu as pltpu
from jax.experimental.pallas import tpu_sc as plsc

BUCKETS = 50
D = 2048
N_ROWS = 16384
LANES = 16

_info = plsc.get_sparse_core_info()
NC, NS = _info.num_cores, _info.num_subcores
NW = NC * NS
ROWS_PER_TILE = N_ROWS // NW
CHUNK = 8
NCHUNK = ROWS_PER_TILE // CHUNK
UNROLL = 4


def _prep_kernel(m1_ref, v1_ref, m2_ref, v2_ref, tab_ref):
    scale = jnp.sqrt(jnp.clip(v2_ref[...] / v1_ref[...], 0.5, 2.0))
    tab_ref[:, :D] = scale
    tab_ref[:, D:] = m2_ref[...] - m1_ref[...] * scale


def _sc_main(labels_hbm, features_hbm, tab_hbm, out_hbm,
             labv, idxv, fa, sa, fb, sb,
             fa_sem, sa_sem, fb_sem, sb_sem, oa_sem, ob_sem):
    wid = lax.axis_index("s") * NC + lax.axis_index("c")
    base = wid * ROWS_PER_TILE
    pltpu.sync_copy(labels_hbm.at[pl.ds(base, ROWS_PER_TILE)], labv)

    def idx_body(j, c):
        lab = labv[pl.ds(j * LANES, LANES)]
        idxv[pl.ds(j * LANES, LANES)] = jnp.where(
            lab <= 1.0, jnp.int32(BUCKETS - 1), jnp.int32(0))
        return c
    lax.fori_loop(0, ROWS_PER_TILE // LANES, idx_body, 0)

    def start_in(c, fbuf, sbuf, fsem, ssem):
        rbase = base + c * CHUNK
        pltpu.async_copy(features_hbm.at[pl.ds(rbase, CHUNK)], fbuf, fsem)
        pltpu.async_copy(tab_hbm.at[idxv.at[pl.ds(c * CHUNK, CHUNK)]],
                         sbuf, ssem)

    def wait_in(c, fbuf, sbuf, fsem, ssem):
        rbase = base + c * CHUNK
        pltpu.make_async_copy(
            features_hbm.at[pl.ds(rbase, CHUNK)], fbuf, fsem).wait()
        pltpu.make_async_copy(
            tab_hbm.at[idxv.at[pl.ds(c * CHUNK, CHUNK)]], sbuf, ssem).wait()

    def start_out(c, fbuf, osem):
        rbase = base + c * CHUNK
        pltpu.async_copy(fbuf, out_hbm.at[pl.ds(rbase, CHUNK)], osem)

    def wait_out(c, fbuf, osem):
        rbase = base + c * CHUNK
        pltpu.make_async_copy(fbuf, out_hbm.at[pl.ds(rbase, CHUNK)],
                              osem).wait()

    def compute(fbuf, sbuf):
        def fma_body(k2, c):
            for u in range(UNROLL):
                col = (k2 * UNROLL + u) * LANES
                sl = pl.ds(col, LANES)
                bsl = pl.ds(D + col, LANES)
                for r in range(CHUNK):
                    fbuf[r, sl] = fbuf[r, sl] * sbuf[r, sl] + sbuf[r, bsl]
            return c
        lax.fori_loop(0, D // (LANES * UNROLL), fma_body, 0)

    start_in(0, fa, sa, fa_sem, sa_sem)
    start_in(1, fb, sb, fb_sem, sb_sem)

    def pair_body(g2, carry):
        c0 = 2 * g2
        c1 = c0 + 1
        wait_in(c0, fa, sa, fa_sem, sa_sem)
        compute(fa, sa)
        start_out(c0, fa, oa_sem)
        wait_in(c1, fb, sb, fb_sem, sb_sem)
        compute(fb, sb)
        start_out(c1, fb, ob_sem)

        def refill(_):
            wait_out(c0, fa, oa_sem)
            start_in(c0 + 2, fa, sa, fa_sem, sa_sem)
            wait_out(c1, fb, ob_sem)
            start_in(c0 + 3, fb, sb, fb_sem, sb_sem)
            return 0

        lax.cond(g2 < NCHUNK // 2 - 1, refill, lambda _: 0, 0)
        return carry

    lax.fori_loop(0, NCHUNK // 2, pair_body, 0)
    wait_out(NCHUNK - 2, fa, oa_sem)
    wait_out(NCHUNK - 1, fb, ob_sem)


@functools.partial(jax.jit, static_argnames=())
def kernel(features, labels, epoch, running_mean_last_epoch,
           running_var_last_epoch, smoothed_mean_last_epoch,
           smoothed_var_last_epoch):
    smooth = epoch >= 1
    m1 = jnp.where(smooth, running_mean_last_epoch, 0.0)
    v1 = jnp.where(smooth, running_var_last_epoch, 1.0)
    m2 = jnp.where(smooth, smoothed_mean_last_epoch, 0.0)
    v2 = jnp.where(smooth, smoothed_var_last_epoch, 1.0)

    tab = pl.pallas_call(
        _prep_kernel,
        out_shape=jax.ShapeDtypeStruct((BUCKETS, 2 * D), jnp.float32),
    )(m1, v1, m2, v2)

    mesh = plsc.VectorSubcoreMesh(core_axis_name="c", subcore_axis_name="s")
    sc = functools.partial(
        pl.kernel, mesh=mesh,
        out_type=jax.ShapeDtypeStruct((N_ROWS, D), jnp.float32),
        scratch_types=[
            pltpu.VMEM((ROWS_PER_TILE,), jnp.float32),
            pltpu.VMEM((ROWS_PER_TILE,), jnp.int32),
            pltpu.VMEM((CHUNK, D), jnp.float32),
            pltpu.VMEM((CHUNK, 2 * D), jnp.float32),
            pltpu.VMEM((CHUNK, D), jnp.float32),
            pltpu.VMEM((CHUNK, 2 * D), jnp.float32),
            pltpu.SemaphoreType.DMA,
            pltpu.SemaphoreType.DMA,
            pltpu.SemaphoreType.DMA,
            pltpu.SemaphoreType.DMA,
            pltpu.SemaphoreType.DMA,
            pltpu.SemaphoreType.DMA,
        ],
    )(_sc_main)
    return sc(labels, features, tab)


# speedup vs baseline: 14.3719x; 14.3719x over previous
import functools

import jax
import jax.numpy as jnp
from jax import lax
from jax.experimental import pallas as pl
from jax.experimental.pallas import tpu as pltpu
from jax.experimental.pallas import tpu_sc as plsc

BUCKETS = 50
D = 2048
N_ROWS = 16384
LANES = 16
BLOCK_N = 1024

_info = plsc.get_sparse_core_info()
NC, NS = _info.num_cores, _info.num_subcores
NW = NC * NS
ROWS_PER_TILE = N_ROWS // NW


def _sc_bucket_idx(labels_hbm, idx_hbm, labv, idxv):
    wid = lax.axis_index("s") * NC + lax.axis_index("c")
    base = wid * ROWS_PER_TILE
    pltpu.sync_copy(labels_hbm.at[pl.ds(base, ROWS_PER_TILE)], labv)

    def idx_body(j, c):
        lab = labv[pl.ds(j * LANES, LANES)]
        idxv[pl.ds(j * LANES, LANES)] = jnp.where(
            lab <= 1.0, jnp.int32(BUCKETS - 1), jnp.int32(0))
        return c
    lax.fori_loop(0, ROWS_PER_TILE // LANES, idx_body, 0)
    pltpu.sync_copy(idxv, idx_hbm.at[pl.ds(base, ROWS_PER_TILE)])


def _tc_main(idx_ref, features_ref, m1_ref, v1_ref, m2_ref, v2_ref,
             out_ref, scale_ref, bias_ref):
    @pl.when(pl.program_id(0) == 0)
    def _prep():
        scale = jnp.sqrt(jnp.clip(v2_ref[...] / v1_ref[...], 0.5, 2.0))
        scale_ref[...] = scale
        bias_ref[...] = m2_ref[...] - m1_ref[...] * scale

    idx = idx_ref[0, 0, :]
    onehot = (idx[:, None] == lax.broadcasted_iota(jnp.int32, (1, BUCKETS), 1)
              ).astype(jnp.float32)
    row_scale = jnp.dot(onehot, scale_ref[...],
                        preferred_element_type=jnp.float32)
    row_bias = jnp.dot(onehot, bias_ref[...],
                       preferred_element_type=jnp.float32)
    out_ref[...] = features_ref[...] * row_scale + row_bias


@functools.partial(jax.jit, static_argnames=())
def kernel(features, labels, epoch, running_mean_last_epoch,
           running_var_last_epoch, smoothed_mean_last_epoch,
           smoothed_var_last_epoch):
    n = features.shape[0]
    grid = n // BLOCK_N
    smooth = epoch >= 1
    m1 = jnp.where(smooth, running_mean_last_epoch, 0.0)
    v1 = jnp.where(smooth, running_var_last_epoch, 1.0)
    m2 = jnp.where(smooth, smoothed_mean_last_epoch, 0.0)
    v2 = jnp.where(smooth, smoothed_var_last_epoch, 1.0)

    mesh = plsc.VectorSubcoreMesh(core_axis_name="c", subcore_axis_name="s")
    idx = functools.partial(
        pl.kernel, mesh=mesh,
        out_type=jax.ShapeDtypeStruct((n,), jnp.int32),
        scratch_types=[
            pltpu.VMEM((ROWS_PER_TILE,), jnp.float32),
            pltpu.VMEM((ROWS_PER_TILE,), jnp.int32),
        ],
    )(_sc_bucket_idx)(labels)
    idx3 = idx.reshape(grid, 1, BLOCK_N)

    table_spec = pl.BlockSpec((BUCKETS, D), lambda i: (0, 0))
    return pl.pallas_call(
        _tc_main,
        grid=(grid,),
        in_specs=[
            pl.BlockSpec((1, 1, BLOCK_N), lambda i: (i, 0, 0)),
            pl.BlockSpec((BLOCK_N, D), lambda i: (i, 0)),
            table_spec, table_spec, table_spec, table_spec,
        ],
        out_specs=pl.BlockSpec((BLOCK_N, D), lambda i: (i, 0)),
        out_shape=jax.ShapeDtypeStruct((n, D), jnp.float32),
        scratch_shapes=[
            pltpu.VMEM((BUCKETS, D), jnp.float32),
            pltpu.VMEM((BUCKETS, D), jnp.float32),
        ],
    )(idx3, features, m1, v1, m2, v2)
